# baseline (speedup 1.0000x reference)
import functools
import jax
import jax.numpy as jnp
from jax import lax
from jax.experimental import pallas as pl
from jax.experimental.pallas import tpu as pltpu
from jax.experimental.pallas import tpu_sc as plsc

_NC = 2
_NS = 16
_NW = _NC * _NS


def _make_sc_add(B, L, D, CH):
    R = B * L
    rows_per_w = R // _NW
    n_chunks = rows_per_w // CH
    CHD = CH * D
    mesh = plsc.VectorSubcoreMesh(core_axis_name="c", subcore_axis_name="s")

    @functools.partial(
        pl.kernel,
        mesh=mesh,
        out_type=jax.ShapeDtypeStruct((R * D,), jnp.float32),
        scratch_types=[
            pltpu.VMEM((CHD,), jnp.float32),
            pltpu.VMEM((CHD,), jnp.float32),
        ],
    )
    def k(x_hbm, table_hbm, out_hbm, xb, tb):
        wid = lax.axis_index("s") * _NC + lax.axis_index("c")
        row0 = wid * rows_per_w
        l0 = lax.rem(row0, L)

        def chunk(c, carry):
            xoff = row0 * D + c * CHD
            toff = l0 * D + c * CHD
            pltpu.sync_copy(x_hbm.at[pl.ds(xoff, CHD)], xb)
            pltpu.sync_copy(table_hbm.at[pl.ds(toff, CHD)], tb)
            for i in range(CHD // 16):
                s = i * 16
                xb[pl.ds(s, 16)] = xb[pl.ds(s, 16)] + tb[pl.ds(s, 16)]
            pltpu.sync_copy(xb, out_hbm.at[pl.ds(xoff, CHD)])
            return carry

        lax.fori_loop(0, n_chunks, chunk, 0)

    return k


def kernel(x, emb_table):
    if x.ndim == 4:
        b, h, l, d = x.shape
        xr = jnp.reshape(jnp.transpose(x, (0, 2, 1, 3)), (b, l, h * d))
        out = kernel(xr, emb_table)
        return jnp.transpose(jnp.reshape(out, (b, l, h, d)), (0, 2, 1, 3))
    B, L, D = x.shape
    xf = jnp.reshape(x, (B * L * D,))
    tf = jnp.reshape(emb_table[:L], (L * D,))
    out = _make_sc_add(B, L, D, 8)(xf, tf)
    return jnp.reshape(out, (B, L, D))


# speedup vs baseline: 7.1541x; 7.1541x over previous
import jax
import jax.numpy as jnp
from jax.experimental import pallas as pl


def _add_kernel(x_ref, emb_ref, o_ref):
    o_ref[...] = x_ref[...] + emb_ref[...][None, :, :]


def _pos_add_3d(x, emb_slice):
    B, L, D = x.shape
    BLK = 128
    grid = (L // BLK,)
    return pl.pallas_call(
        _add_kernel,
        grid=grid,
        in_specs=[
            pl.BlockSpec((B, BLK, D), lambda i: (0, i, 0)),
            pl.BlockSpec((BLK, D), lambda i: (i, 0)),
        ],
        out_specs=pl.BlockSpec((B, BLK, D), lambda i: (0, i, 0)),
        out_shape=jax.ShapeDtypeStruct((B, L, D), x.dtype),
    )(x, emb_slice)


def kernel(x, emb_table):
    if x.ndim == 3:
        L = x.shape[-2]
        return _pos_add_3d(x, emb_table[:L])
    b, h, l, d = x.shape
    xr = jnp.reshape(jnp.transpose(x, (0, 2, 1, 3)), (b, l, h * d))
    xr = _pos_add_3d(xr, emb_table[:l])
    return jnp.transpose(jnp.reshape(xr, (b, l, h, d)), (0, 2, 1, 3))
